# Initial kernel scaffold; baseline (speedup 1.0000x reference)
import jax
import jax.numpy as jnp
from jax import lax
from jax.experimental import pallas as pl
from jax.experimental.pallas import tpu as pltpu
from jax.experimental.pallas import tpu_sc as plsc

N_NODES = 50000
D_FEAT = 256
HOP_LEN = 256

NC = 2
NS = 16
L = 16

ROWS_PER = HOP_LEN // NS
NCHUNK = D_FEAT // L


def _body(gcn_hbm, rawx_hbm, w_hbm, hop_hbm, out0_hbm, out1_hbm,
          idx_v, sv_v, rows_v, part_v, shared, sum_v, out_v, sem):
    c = lax.axis_index("c")
    s = lax.axis_index("s")
    base = s * ROWS_PER

    pltpu.sync_copy(hop_hbm.at[pl.ds(base, ROWS_PER)], idx_v)
    pltpu.sync_copy(w_hbm.at[pl.ds(base, ROWS_PER)], sv_v)
    wv = sv_v[...]
    sv_v[...] = (1.0 / (1.0 + jnp.exp(-wv))) * (1.0 / HOP_LEN)

    def gather_and_accumulate(mat_hbm):
        pltpu.async_copy(mat_hbm.at[idx_v], rows_v, sem).wait()
        for k in range(NCHUNK):
            acc = jnp.zeros((L,), jnp.float32)
            for j in range(ROWS_PER):
                acc = acc + sv_v[j] * rows_v[j, pl.ds(k * L, L)]
            part_v[pl.ds(k * L, L)] = acc

    @pl.when(c == 0)
    def _():
        gather_and_accumulate(gcn_hbm)

    @pl.when(c == 1)
    def _():
        gather_and_accumulate(rawx_hbm)

    pltpu.sync_copy(part_v, shared.at[s])
    plsc.subcore_barrier()

    @pl.when(s == 0)
    def _():
        pltpu.sync_copy(shared, sum_v)
        for k in range(NCHUNK):
            acc = jnp.zeros((L,), jnp.float32)
            for r in range(NS):
                acc = acc + sum_v[r, pl.ds(k * L, L)]
            out_v[pl.ds(k * L, L)] = acc

        @pl.when(c == 0)
        def _():
            pltpu.sync_copy(out_v, out0_hbm)

        @pl.when(c == 1)
        def _():
            pltpu.sync_copy(out_v, out1_hbm)


_sc_call = pl.kernel(
    _body,
    out_type=(
        jax.ShapeDtypeStruct((D_FEAT,), jnp.float32),
        jax.ShapeDtypeStruct((D_FEAT,), jnp.float32),
    ),
    mesh=plsc.VectorSubcoreMesh(core_axis_name="c", subcore_axis_name="s"),
    scratch_types=[
        pltpu.VMEM((ROWS_PER,), jnp.int32),
        pltpu.VMEM((ROWS_PER,), jnp.float32),
        pltpu.VMEM((ROWS_PER, D_FEAT), jnp.float32),
        pltpu.VMEM((D_FEAT,), jnp.float32),
        pltpu.VMEM_SHARED((NS, D_FEAT), jnp.float32),
        pltpu.VMEM((NS, D_FEAT), jnp.float32),
        pltpu.VMEM((D_FEAT,), jnp.float32),
        pltpu.SemaphoreType.DMA,
    ],
)


@jax.jit
def kernel(gcn_features, rawX, weight, hop):
    out, proxy = _sc_call(gcn_features, rawX, weight.reshape(HOP_LEN), hop)
    return (out, proxy)


# speedup vs baseline: 1.2903x; 1.2903x over previous
import jax
import jax.numpy as jnp
from jax import lax
from jax.experimental import pallas as pl
from jax.experimental.pallas import tpu as pltpu
from jax.experimental.pallas import tpu_sc as plsc

N_NODES = 50000
D_FEAT = 256
HOP_LEN = 256

NS = 16
L = 16

ROWS_PER = HOP_LEN // NS
NCHUNK = D_FEAT // L


def _body(gcn_hbm, rawx_hbm, w_hbm, hop_hbm, out0_hbm, out1_hbm,
          idx_v, sv_v, rows_v, part_v, shared, sum_v, out_v, sem):
    c = lax.axis_index("c")
    s = lax.axis_index("s")

    @pl.when(c == 0)
    def _():
        base = s * ROWS_PER

        pltpu.sync_copy(hop_hbm.at[pl.ds(base, ROWS_PER)], idx_v)
        pltpu.sync_copy(w_hbm.at[pl.ds(base, ROWS_PER)], sv_v)
        sv = (1.0 / (1.0 + jnp.exp(-sv_v[...]))) * (1.0 / HOP_LEN)

        cp0 = pltpu.async_copy(gcn_hbm.at[idx_v], rows_v.at[0], sem)
        cp1 = pltpu.async_copy(rawx_hbm.at[idx_v], rows_v.at[1], sem)
        cp0.wait()
        cp1.wait()

        for m in range(2):
            for k in range(NCHUNK):
                acc = jnp.zeros((L,), jnp.float32)
                for j in range(ROWS_PER):
                    acc = acc + sv[j] * rows_v[m, j, pl.ds(k * L, L)]
                part_v[m, pl.ds(k * L, L)] = acc

        pltpu.sync_copy(part_v, shared.at[s])
        plsc.subcore_barrier()

        @pl.when(s == 0)
        def _():
            pltpu.sync_copy(shared, sum_v)
            for m in range(2):
                for k in range(NCHUNK):
                    acc = jnp.zeros((L,), jnp.float32)
                    for r in range(NS):
                        acc = acc + sum_v[r, m, pl.ds(k * L, L)]
                    out_v[m, pl.ds(k * L, L)] = acc
            pltpu.sync_copy(out_v.at[0], out0_hbm)
            pltpu.sync_copy(out_v.at[1], out1_hbm)


_sc_call = pl.kernel(
    _body,
    out_type=(
        jax.ShapeDtypeStruct((D_FEAT,), jnp.float32),
        jax.ShapeDtypeStruct((D_FEAT,), jnp.float32),
    ),
    mesh=plsc.VectorSubcoreMesh(core_axis_name="c", subcore_axis_name="s"),
    scratch_types=[
        pltpu.VMEM((ROWS_PER,), jnp.int32),
        pltpu.VMEM((ROWS_PER,), jnp.float32),
        pltpu.VMEM((2, ROWS_PER, D_FEAT), jnp.float32),
        pltpu.VMEM((2, D_FEAT), jnp.float32),
        pltpu.VMEM_SHARED((NS, 2, D_FEAT), jnp.float32),
        pltpu.VMEM((NS, 2, D_FEAT), jnp.float32),
        pltpu.VMEM((2, D_FEAT), jnp.float32),
        pltpu.SemaphoreType.DMA,
    ],
)


@jax.jit
def kernel(gcn_features, rawX, weight, hop):
    out, proxy = _sc_call(gcn_features, rawX, weight.reshape(HOP_LEN), hop)
    return (out, proxy)


# speedup vs baseline: 1.5335x; 1.1885x over previous
import jax
import jax.numpy as jnp
from jax import lax
from jax.experimental import pallas as pl
from jax.experimental.pallas import tpu as pltpu
from jax.experimental.pallas import tpu_sc as plsc

N_NODES = 50000
D_FEAT = 256
HOP_LEN = 256

NS = 16
L = 16

ROWS_PER = HOP_LEN // NS
NCHUNK = D_FEAT // L


def _body(gcn_hbm, rawx_hbm, w_hbm, hop_hbm, out0_hbm, out1_hbm,
          idx_v, sv_v, rows_v, part_v, shared, red_v, out_v, sem):
    s = lax.axis_index("s")
    base = s * ROWS_PER

    pltpu.sync_copy(hop_hbm.at[pl.ds(base, ROWS_PER)], idx_v)
    cp0 = pltpu.async_copy(gcn_hbm.at[idx_v], rows_v.at[0], sem)
    cp1 = pltpu.async_copy(rawx_hbm.at[idx_v], rows_v.at[1], sem)
    pltpu.sync_copy(w_hbm.at[pl.ds(base, ROWS_PER)], sv_v)
    sv = (1.0 / (1.0 + jnp.exp(-sv_v[...]))) * (1.0 / HOP_LEN)
    cp0.wait()
    cp1.wait()

    for m in range(2):
        for k in range(NCHUNK):
            acc = jnp.zeros((L,), jnp.float32)
            for j in range(ROWS_PER):
                acc = acc + sv[j] * rows_v[m, j, pl.ds(k * L, L)]
            part_v[m, pl.ds(k * L, L)] = acc

    pltpu.sync_copy(part_v, shared.at[s])
    plsc.subcore_barrier()

    @pl.when(s == 0)
    def _():
        pltpu.sync_copy(shared, red_v)
        for m in range(2):
            for k in range(NCHUNK):
                acc = jnp.zeros((L,), jnp.float32)
                for r in range(NS):
                    acc = acc + red_v[r, m, pl.ds(k * L, L)]
                out_v[m, pl.ds(k * L, L)] = acc
        pltpu.sync_copy(out_v.at[0], out0_hbm)
        pltpu.sync_copy(out_v.at[1], out1_hbm)


_sc_call = pl.kernel(
    _body,
    out_type=(
        jax.ShapeDtypeStruct((D_FEAT,), jnp.float32),
        jax.ShapeDtypeStruct((D_FEAT,), jnp.float32),
    ),
    mesh=plsc.VectorSubcoreMesh(
        core_axis_name="c", subcore_axis_name="s", num_cores=1),
    scratch_types=[
        pltpu.VMEM((ROWS_PER,), jnp.int32),
        pltpu.VMEM((ROWS_PER,), jnp.float32),
        pltpu.VMEM((2, ROWS_PER, D_FEAT), jnp.float32),
        pltpu.VMEM((2, D_FEAT), jnp.float32),
        pltpu.VMEM_SHARED((NS, 2, D_FEAT), jnp.float32),
        pltpu.VMEM((NS, 2, D_FEAT), jnp.float32),
        pltpu.VMEM((2, D_FEAT), jnp.float32),
        pltpu.SemaphoreType.DMA,
    ],
)


@jax.jit
def kernel(gcn_features, rawX, weight, hop):
    out, proxy = _sc_call(gcn_features, rawX, weight.reshape(HOP_LEN), hop)
    return (out, proxy)
